# baseline (speedup 1.0000x reference)
import jax
import jax.numpy as jnp
from jax.experimental import pallas as pl
from jax.experimental.pallas import tpu as pltpu

_BLK = 2048
_L = 256


def _body(pay_ref, pert_ref, emb_hbm, out_ref, sem):
    i = pl.program_id(0)
    b, s, d = emb_hbm.shape
    nj = s // _BLK
    bi = i // nj
    j = i % nj
    cp = pltpu.make_async_copy(
        emb_hbm.at[pl.ds(bi, 1), pl.ds(j * _BLK, _BLK), :],
        out_ref,
        sem,
    )
    cp.start()
    cp.wait()

    @pl.when(j == 0)
    def _():
        mask = (pay_ref[...] == 1).astype(out_ref.dtype)
        out_ref[0, :_L, :] = out_ref[0, :_L, :] + pert_ref[...] * mask


def kernel(embedded_input, watermark_payload, perturbation_vectors):
    b, s, d = embedded_input.shape
    l = perturbation_vectors.shape[0]
    nj = s // _BLK
    pay2d = watermark_payload.reshape(l, 1)
    return pl.pallas_call(
        _body,
        grid=(b * nj,),
        in_specs=[
            pl.BlockSpec((l, 1), lambda i: (0, 0)),
            pl.BlockSpec((l, d), lambda i: (0, 0)),
            pl.BlockSpec(memory_space=pltpu.MemorySpace.HBM),
        ],
        out_specs=pl.BlockSpec((1, _BLK, d), lambda i: (i // (4096 // _BLK), i % (4096 // _BLK), 0)),
        out_shape=jax.ShapeDtypeStruct((b, s, d), embedded_input.dtype),
        scratch_shapes=[
            pltpu.SemaphoreType.DMA,
        ],
    )(pay2d, perturbation_vectors, embedded_input)


# speedup vs baseline: 1.1204x; 1.1204x over previous
import jax
import jax.numpy as jnp
from jax.experimental import pallas as pl
from jax.experimental.pallas import tpu as pltpu

_BLK = 1024
_L = 256
_R = 4


def _body(pay_ref, pert_ref, emb_hbm, out_hbm, ring, isem, osem):
    b, s, d = emb_hbm.shape
    nj = s // _BLK
    n = b * nj

    def in_copy(k):
        bi, j = k // nj, k % nj
        sl = k % _R
        return pltpu.make_async_copy(
            emb_hbm.at[pl.ds(bi, 1), pl.ds(j * _BLK, _BLK), :],
            ring.at[pl.ds(sl, 1)],
            isem.at[sl],
        )

    def out_copy(k):
        bi, j = k // nj, k % nj
        sl = k % _R
        return pltpu.make_async_copy(
            ring.at[pl.ds(sl, 1)],
            out_hbm.at[pl.ds(bi, 1), pl.ds(j * _BLK, _BLK), :],
            osem.at[sl],
        )

    for k in range(min(_R - 1, n)):
        in_copy(k).start()

    mask = (pay_ref[...] == 1).astype(ring.dtype)
    pert = pert_ref[...] * mask

    for k in range(n):
        sl = k % _R
        in_copy(k).wait()
        if k % nj == 0:
            ring[sl, :_L, :] = ring[sl, :_L, :] + pert
        out_copy(k).start()
        nk = k + _R - 1
        if nk < n:
            if k >= 1:
                out_copy(k - 1).wait()
            in_copy(nk).start()
    waited = set(k - 1 for k in range(1, n) if (k + _R - 1) < n)
    for k in range(n):
        if k not in waited:
            out_copy(k).wait()


def kernel(embedded_input, watermark_payload, perturbation_vectors):
    b, s, d = embedded_input.shape
    l = perturbation_vectors.shape[0]
    pay2d = watermark_payload.reshape(l, 1)
    return pl.pallas_call(
        _body,
        in_specs=[
            pl.BlockSpec((l, 1), lambda: (0, 0)),
            pl.BlockSpec((l, d), lambda: (0, 0)),
            pl.BlockSpec(memory_space=pltpu.MemorySpace.HBM),
        ],
        out_specs=pl.BlockSpec(memory_space=pltpu.MemorySpace.HBM),
        out_shape=jax.ShapeDtypeStruct((b, s, d), embedded_input.dtype),
        scratch_shapes=[
            pltpu.VMEM((_R, _BLK, d), embedded_input.dtype),
            pltpu.SemaphoreType.DMA((_R,)),
            pltpu.SemaphoreType.DMA((_R,)),
        ],
    )(pay2d, perturbation_vectors, embedded_input)
